# Initial kernel scaffold; baseline (speedup 1.0000x reference)
import jax
import jax.numpy as jnp
from jax.experimental import pallas as pl


def kernel(z, weight):
    raise NotImplementedError("write your pallas kernel here")


# speedup vs baseline: 7.8187x; 7.8187x over previous
import functools

import jax
import jax.numpy as jnp
from jax import lax
from jax.experimental import pallas as pl
from jax.experimental.pallas import tpu as pltpu
from jax.experimental.pallas import tpu_sc as plsc

_S, _K, _D = 8, 512, 8
_NB = 1024
_CH = 128


def _argmin_body(z_ref, w_ref, idx_ref):
    z = z_ref[...]
    w = w_ref[...]
    cols = []
    for s in range(_S):
        zs = z[:, s * _D:(s + 1) * _D]
        ws = w[s]
        cross = lax.dot_general(
            zs, ws, (((1,), (1,)), ((), ())),
            preferred_element_type=jnp.float32)
        zz = jnp.sum(zs * zs, axis=1, keepdims=True)
        ww = jnp.sum(ws * ws, axis=1)[None, :]
        d2 = jnp.maximum(zz + ww - 2.0 * cross, 0.0)
        minv = jnp.min(d2, axis=1, keepdims=True)
        kio = lax.broadcasted_iota(jnp.int32, d2.shape, 1)
        idx = jnp.min(jnp.where(d2 == minv, kio, _K), axis=1) + s * _K
        cols.append(idx[:, None])
    idx_ref[...] = jnp.concatenate(cols, axis=1)


def _tc_indices(z2, weight):
    n = z2.shape[0]
    return pl.pallas_call(
        _argmin_body,
        grid=(n // _NB,),
        in_specs=[
            pl.BlockSpec((_NB, _S * _D), lambda i: (i, 0)),
            pl.BlockSpec((_S, _K, _D), lambda i: (0, 0, 0)),
        ],
        out_specs=pl.BlockSpec((_NB, _S), lambda i: (i, 0)),
        out_shape=jax.ShapeDtypeStruct((n, _S), jnp.int32),
    )(z2, weight)


def _gather_body(table_hbm, idx_hbm, out_hbm, idx_v, rows_v, sem):
    nch = idx_v.shape[0]
    bpw = nch * _CH
    wid = lax.axis_index("s") * 2 + lax.axis_index("c")
    pltpu.sync_copy(idx_hbm.at[pl.ds(wid * nch, nch)], idx_v)

    def grp(g, carry):
        h = [pltpu.async_copy(table_hbm.at[idx_v.at[g * 8 + j]],
                              rows_v.at[pl.ds((g * 8 + j) * _CH, _CH)], sem)
             for j in range(8)]
        for c in h:
            c.wait()
        return carry
    lax.fori_loop(0, nch // 8, grp, 0)
    pltpu.sync_copy(rows_v, out_hbm.at[pl.ds(wid * bpw, bpw)])


def _sc_gather(table, idx2):
    nw = 32
    b = idx2.shape[0] * idx2.shape[1]
    nch = b // (nw * _CH)
    fn = functools.partial(
        pl.kernel,
        mesh=plsc.VectorSubcoreMesh(core_axis_name="c", subcore_axis_name="s"),
        out_type=jax.ShapeDtypeStruct((b, _D), jnp.float32),
        scratch_types=[
            pltpu.VMEM((nch, _CH), jnp.int32),
            pltpu.VMEM((nch * _CH, _D), jnp.float32),
            pltpu.SemaphoreType.DMA,
        ],
        compiler_params=pltpu.CompilerParams(use_tc_tiling_on_sc=False),
    )(_gather_body)
    return fn(table, idx2)


def kernel(z, weight):
    zshape = z.shape
    z2 = z.reshape(-1, _S * _D)
    gidx = _tc_indices(z2, weight)
    table = weight.reshape(_S * _K, _D)
    idx2 = gidx.reshape(-1, _CH)
    out = _sc_gather(table, idx2)
    return out.reshape(zshape)


# speedup vs baseline: 10.6664x; 1.3642x over previous
import functools

import jax
import jax.numpy as jnp
from jax import lax
from jax.experimental import pallas as pl
from jax.experimental.pallas import tpu as pltpu
from jax.experimental.pallas import tpu_sc as plsc

_S, _K, _D = 8, 512, 8
_NB = 1024
_CH = 128


def _argmin_body(z_ref, w_ref, idx_ref):
    z = z_ref[...]
    w = w_ref[...]
    scores = []
    for s in range(_S):
        zs = z[:, s * _D:(s + 1) * _D]
        ws = w[s]
        cross = lax.dot_general(
            zs, ws, (((1,), (1,)), ((), ())),
            preferred_element_type=jnp.float32)
        ww = jnp.sum(ws * ws, axis=1)[None, :]
        scores.append(2.0 * cross - ww)
    cols = [jnp.argmax(sc, axis=1).astype(jnp.int32)[:, None] + s * _K
            for s, sc in enumerate(scores)]
    idx_ref[...] = jnp.concatenate(cols, axis=1)


def _tc_indices(z2, weight):
    n = z2.shape[0]
    return pl.pallas_call(
        _argmin_body,
        grid=(n // _NB,),
        in_specs=[
            pl.BlockSpec((_NB, _S * _D), lambda i: (i, 0)),
            pl.BlockSpec((_S, _K, _D), lambda i: (0, 0, 0)),
        ],
        out_specs=pl.BlockSpec((_NB, _S), lambda i: (i, 0)),
        out_shape=jax.ShapeDtypeStruct((n, _S), jnp.int32),
    )(z2, weight)


def _gather_body(table_hbm, idx_hbm, out_hbm, idx_v, rows_v, sem):
    nch = idx_v.shape[0]
    bpw = nch * _CH
    wid = lax.axis_index("s") * 2 + lax.axis_index("c")
    pltpu.sync_copy(idx_hbm.at[pl.ds(wid * nch, nch)], idx_v)

    def grp(g, carry):
        h = [pltpu.async_copy(table_hbm.at[idx_v.at[g * 8 + j]],
                              rows_v.at[pl.ds((g * 8 + j) * _CH, _CH)], sem)
             for j in range(8)]
        for c in h:
            c.wait()
        return carry
    lax.fori_loop(0, nch // 8, grp, 0)
    pltpu.sync_copy(rows_v, out_hbm.at[pl.ds(wid * bpw, bpw)])


def _sc_gather(table, idx2):
    nw = 32
    b = idx2.shape[0] * idx2.shape[1]
    nch = b // (nw * _CH)
    fn = functools.partial(
        pl.kernel,
        mesh=plsc.VectorSubcoreMesh(core_axis_name="c", subcore_axis_name="s"),
        out_type=jax.ShapeDtypeStruct((b, _D), jnp.float32),
        scratch_types=[
            pltpu.VMEM((nch, _CH), jnp.int32),
            pltpu.VMEM((nch * _CH, _D), jnp.float32),
            pltpu.SemaphoreType.DMA,
        ],
        compiler_params=pltpu.CompilerParams(use_tc_tiling_on_sc=False),
    )(_gather_body)
    return fn(table, idx2)


def kernel(z, weight):
    zshape = z.shape
    z2 = z.reshape(-1, _S * _D)
    gidx = _tc_indices(z2, weight)
    table = weight.reshape(_S * _K, _D)
    idx2 = gidx.reshape(-1, _CH)
    out = _sc_gather(table, idx2)
    return out.reshape(zshape)


# speedup vs baseline: 11.2752x; 1.0571x over previous
import functools

import jax
import jax.numpy as jnp
from jax import lax
from jax.experimental import pallas as pl
from jax.experimental.pallas import tpu as pltpu
from jax.experimental.pallas import tpu_sc as plsc

_S, _K, _D = 8, 512, 8
_NB = 2048
_CH = 128


def _argmin_body(z_ref, w_ref, idx_ref):
    z = z_ref[...]
    w = w_ref[...]
    scores = []
    for s in range(_S):
        zs = z[:, s * _D:(s + 1) * _D]
        ws = w[s]
        cross = lax.dot_general(
            zs, ws, (((1,), (1,)), ((), ())),
            preferred_element_type=jnp.float32)
        ww = jnp.sum(ws * ws, axis=1)[None, :]
        scores.append(2.0 * cross - ww)
    cols = [jnp.argmax(sc, axis=1).astype(jnp.int32)[:, None] + s * _K
            for s, sc in enumerate(scores)]
    idx_ref[...] = jnp.concatenate(cols, axis=1)


def _tc_indices(z2, weight):
    n = z2.shape[0]
    return pl.pallas_call(
        _argmin_body,
        grid=(n // _NB,),
        in_specs=[
            pl.BlockSpec((_NB, _S * _D), lambda i: (i, 0)),
            pl.BlockSpec((_S, _K, _D), lambda i: (0, 0, 0)),
        ],
        out_specs=pl.BlockSpec((_NB, _S), lambda i: (i, 0)),
        out_shape=jax.ShapeDtypeStruct((n, _S), jnp.int32),
    )(z2, weight)


def _gather_body(table_hbm, idx_hbm, out_hbm, idx_v, rows_v, sem):
    nch = idx_v.shape[0]
    bpw = nch * _CH
    wid = lax.axis_index("s") * 2 + lax.axis_index("c")
    pltpu.sync_copy(idx_hbm.at[pl.ds(wid * nch, nch)], idx_v)

    def grp(g, carry):
        h = [pltpu.async_copy(table_hbm.at[idx_v.at[g * 8 + j]],
                              rows_v.at[pl.ds((g * 8 + j) * _CH, _CH)], sem)
             for j in range(8)]
        for c in h:
            c.wait()
        return carry
    lax.fori_loop(0, nch // 8, grp, 0)
    pltpu.sync_copy(rows_v, out_hbm.at[pl.ds(wid * bpw, bpw)])


def _sc_gather(table, idx2):
    nw = 32
    b = idx2.shape[0] * idx2.shape[1]
    nch = b // (nw * _CH)
    fn = functools.partial(
        pl.kernel,
        mesh=plsc.VectorSubcoreMesh(core_axis_name="c", subcore_axis_name="s"),
        out_type=jax.ShapeDtypeStruct((b, _D), jnp.float32),
        scratch_types=[
            pltpu.VMEM((nch, _CH), jnp.int32),
            pltpu.VMEM((nch * _CH, _D), jnp.float32),
            pltpu.SemaphoreType.DMA,
        ],
        compiler_params=pltpu.CompilerParams(use_tc_tiling_on_sc=False),
    )(_gather_body)
    return fn(table, idx2)


def kernel(z, weight):
    zshape = z.shape
    z2 = z.reshape(-1, _S * _D)
    gidx = _tc_indices(z2, weight)
    table = weight.reshape(_S * _K, _D)
    idx2 = gidx.reshape(-1, _CH)
    out = _sc_gather(table, idx2)
    return out.reshape(zshape)
